# baseline (speedup 1.0000x reference)
import jax
import jax.numpy as jnp
from jax import lax
from jax.experimental import pallas as pl
from jax.experimental.pallas import tpu as pltpu
from jax.experimental.pallas import tpu_sc as plsc

NC = 2
NS = 16
NW = NC * NS
LANES = 16
NCH = 11
NKEEP = 3
NT = 288
D = 64
N = 512
B = 16
G8 = 8
BPW = B * NT // NW


GRP = 8
NRING = 8
DH = D // 4


def _sc_body(x_rows, tab_hbm, ebd_hbm, tabv, xbuf, ebuf0, ebuf1, ebuf2,
             ebuf3, ebuf4, ebuf5, ebuf6, ebuf7,
             sem0, sem1, sem2, sem3, sem4, sem5, sem6, sem7):
    wid = lax.axis_index("s") * NC + lax.axis_index("c")
    b = wid >> 1
    t0 = (wid & 1) * BPW

    pltpu.sync_copy(tab_hbm, tabv)

    xrow0 = b * (NCH * NT) + 3 * NT + t0
    bt0 = b * NT + t0
    ebufs = (ebuf0, ebuf1, ebuf2, ebuf3, ebuf4, ebuf5, ebuf6, ebuf7)
    sems = (sem0, sem1, sem2, sem3, sem4, sem5, sem6, sem7)

    def i_body(i, carry):
        pltpu.sync_copy(x_rows.at[pl.ds(xrow0 + i * GRP, GRP), :], xbuf)
        for r in range(GRP):
            for h in range(4):
                k = r * 4 + h
                ebuf = ebufs[k % NRING]
                sem = sems[k % NRING]

                if k >= NRING:
                    pltpu.make_async_copy(
                        ebuf, ebd_hbm.at[pl.ds(0, DH), :], sem).wait()
                else:
                    @pl.when(i > 0)
                    def _():
                        pltpu.make_async_copy(
                            ebuf, ebd_hbm.at[pl.ds(0, DH), :], sem).wait()

                @plsc.parallel_loop(0, N // LANES, step=1)
                def j_loop(j):
                    v = xbuf[r, pl.ds(j * LANES, LANES)]
                    base = (v * 288.0).astype(jnp.int32) + h * (DH * NT)
                    for d in range(DH):
                        g = plsc.load_gather(tabv, [base + d * NT])
                        ebuf[d, pl.ds(j * LANES, LANES)] = g

                out_row = (bt0 + i * GRP + r) * D + h * DH
                pltpu.async_copy(ebuf, ebd_hbm.at[pl.ds(out_row, DH), :], sem)
        return carry

    lax.fori_loop(0, BPW // GRP, i_body, 0)
    for q in range(NRING):
        pltpu.make_async_copy(ebufs[q], ebd_hbm.at[pl.ds(0, DH), :], sems[q]).wait()


def _tc_slice_body(x_ref, o_ref):
    o_ref[...] = x_ref[...]


@jax.jit
def kernel(x, time_table):
    b, n, t, ch = x.shape
    assert (b, n, t, ch) == (B, N, NT, NCH) and time_table.shape == (NT, D)

    xt = jnp.transpose(x, (0, 3, 2, 1))
    x_rows = xt.reshape(b * ch * t, n)
    tab_flat = time_table.T.reshape(NT * D)

    mesh = plsc.VectorSubcoreMesh(core_axis_name="c", subcore_axis_name="s")
    ebd_rows = pl.kernel(
        _sc_body,
        out_type=jax.ShapeDtypeStruct((b * t * D, n), jnp.float32),
        mesh=mesh,
        compiler_params=pltpu.CompilerParams(
            needs_layout_passes=False, use_tc_tiling_on_sc=True),
        scratch_types=[
            pltpu.VMEM((NT * D,), jnp.float32),
            pltpu.VMEM((GRP, N), jnp.float32),
            *[pltpu.VMEM((DH, N), jnp.float32) for _ in range(NRING)],
            *[pltpu.SemaphoreType.DMA for _ in range(NRING)],
        ],
    )(x_rows, tab_flat)

    xo_t = pl.pallas_call(
        _tc_slice_body,
        grid=(b, NKEEP),
        in_specs=[pl.BlockSpec((1, 1, t, n), lambda i, c: (i, c, 0, 0))],
        out_specs=pl.BlockSpec((1, 1, t, n), lambda i, c: (i, c, 0, 0)),
        out_shape=jax.ShapeDtypeStruct((b, NKEEP, t, n), jnp.float32),
    )(xt)

    xo = jnp.transpose(xo_t, (0, 3, 2, 1))
    ebd = jnp.transpose(ebd_rows.reshape(b, t, D, n), (0, 3, 1, 2))
    return xo, ebd


# speedup vs baseline: 1.0616x; 1.0616x over previous
import jax
import jax.numpy as jnp
from jax import lax
from jax.experimental import pallas as pl
from jax.experimental.pallas import tpu as pltpu
from jax.experimental.pallas import tpu_sc as plsc

NC = 2
NS = 16
NW = NC * NS
LANES = 16
NCH = 11
NKEEP = 3
NT = 288
D = 64
N = 512
B = 16
G8 = 8
BPW = B * NT // NW


GRP = 8
NRING = 4
DH = D // 2


def _sc_body(x_rows, tab_hbm, ebd_hbm, tabv, xbuf, ebuf0, ebuf1, ebuf2,
             ebuf3, sem0, sem1, sem2, sem3):
    wid = lax.axis_index("s") * NC + lax.axis_index("c")
    b = wid >> 1
    t0 = (wid & 1) * BPW

    pltpu.sync_copy(tab_hbm, tabv)

    xrow0 = b * (NCH * NT) + 3 * NT + t0
    bt0 = b * NT + t0
    ebufs = (ebuf0, ebuf1, ebuf2, ebuf3)
    sems = (sem0, sem1, sem2, sem3)

    def i_body(i, carry):
        pltpu.sync_copy(x_rows.at[pl.ds(xrow0 + i * GRP, GRP), :], xbuf)
        for r in range(GRP):
            for h in range(2):
                k = r * 2 + h
                ebuf = ebufs[k % NRING]
                sem = sems[k % NRING]

                if k >= NRING:
                    pltpu.make_async_copy(
                        ebuf, ebd_hbm.at[pl.ds(0, DH), :], sem).wait()
                else:
                    @pl.when(i > 0)
                    def _():
                        pltpu.make_async_copy(
                            ebuf, ebd_hbm.at[pl.ds(0, DH), :], sem).wait()

                @plsc.parallel_loop(0, N // LANES, step=1)
                def j_loop(j):
                    v = xbuf[r, pl.ds(j * LANES, LANES)]
                    base = (v * 288.0).astype(jnp.int32) + h * (DH * NT)
                    for d in range(DH):
                        g = plsc.load_gather(tabv, [base + d * NT])
                        ebuf[d, pl.ds(j * LANES, LANES)] = g

                out_row = (bt0 + i * GRP + r) * D + h * DH
                pltpu.async_copy(ebuf, ebd_hbm.at[pl.ds(out_row, DH), :], sem)
        return carry

    lax.fori_loop(0, BPW // GRP, i_body, 0)
    for q in range(NRING):
        pltpu.make_async_copy(ebufs[q], ebd_hbm.at[pl.ds(0, DH), :], sems[q]).wait()


def _tc_slice_body(x_ref, o_ref):
    o_ref[...] = x_ref[...]


@jax.jit
def kernel(x, time_table):
    b, n, t, ch = x.shape
    assert (b, n, t, ch) == (B, N, NT, NCH) and time_table.shape == (NT, D)

    xt = jnp.transpose(x, (0, 3, 2, 1))
    x_rows = xt.reshape(b * ch * t, n)
    tab_flat = time_table.T.reshape(NT * D)

    mesh = plsc.VectorSubcoreMesh(core_axis_name="c", subcore_axis_name="s")
    ebd_rows = pl.kernel(
        _sc_body,
        out_type=jax.ShapeDtypeStruct((b * t * D, n), jnp.float32),
        mesh=mesh,
        compiler_params=pltpu.CompilerParams(
            needs_layout_passes=False, use_tc_tiling_on_sc=True),
        scratch_types=[
            pltpu.VMEM((NT * D,), jnp.float32),
            pltpu.VMEM((GRP, N), jnp.float32),
            pltpu.VMEM((DH, N), jnp.float32),
            pltpu.VMEM((DH, N), jnp.float32),
            pltpu.VMEM((DH, N), jnp.float32),
            pltpu.VMEM((DH, N), jnp.float32),
            pltpu.SemaphoreType.DMA,
            pltpu.SemaphoreType.DMA,
            pltpu.SemaphoreType.DMA,
            pltpu.SemaphoreType.DMA,
        ],
    )(x_rows, tab_flat)

    xo_t = pl.pallas_call(
        _tc_slice_body,
        grid=(b, NKEEP),
        in_specs=[pl.BlockSpec((1, 1, t, n), lambda i, c: (i, c, 0, 0))],
        out_specs=pl.BlockSpec((1, 1, t, n), lambda i, c: (i, c, 0, 0)),
        out_shape=jax.ShapeDtypeStruct((b, NKEEP, t, n), jnp.float32),
    )(xt)

    xo = jnp.transpose(xo_t, (0, 3, 2, 1))
    ebd = jnp.transpose(ebd_rows.reshape(b, t, D, n), (0, 3, 1, 2))
    return xo, ebd


# speedup vs baseline: 1.0935x; 1.0300x over previous
import jax
import jax.numpy as jnp
from jax import lax
from jax.experimental import pallas as pl
from jax.experimental.pallas import tpu as pltpu
from jax.experimental.pallas import tpu_sc as plsc

NC = 2
NS = 16
NW = NC * NS
LANES = 16
NCH = 11
NKEEP = 3
NT = 288
D = 64
N = 512
B = 16
G8 = 8
BPW = B * NT // NW


GRP = 8
NRING = 4
DH = D // 2


def _sc_body(x_rows, tab_hbm, ebd_hbm, tabv, xbuf, ebuf0, ebuf1, ebuf2,
             ebuf3, sem0, sem1, sem2, sem3, isem):
    wid = lax.axis_index("s") * NC + lax.axis_index("c")
    b = wid >> 1
    t0 = (wid & 1) * BPW

    pltpu.sync_copy(tab_hbm, tabv)

    xrow0 = b * (NCH * NT) + 3 * NT + t0
    bt0 = b * NT + t0
    ebufs = (ebuf0, ebuf1, ebuf2, ebuf3)
    sems = (sem0, sem1, sem2, sem3)
    n_i = BPW // GRP

    pltpu.sync_copy(x_rows.at[pl.ds(xrow0, GRP), :],
                    xbuf.at[pl.ds(0, GRP), :])

    def i_body(i, carry):
        @pl.when(i > 0)
        def _():
            pltpu.make_async_copy(
                x_rows.at[pl.ds(xrow0, GRP), :],
                xbuf.at[pl.ds(0, GRP), :], isem).wait()

        @pl.when(i < n_i - 1)
        def _():
            pltpu.async_copy(
                x_rows.at[pl.ds(xrow0 + (i + 1) * GRP, GRP), :],
                xbuf.at[pl.ds(((i + 1) & 1) * GRP, GRP), :], isem)

        xrow = (i & 1) * GRP
        for r in range(GRP):
            for h in range(2):
                k = r * 2 + h
                ebuf = ebufs[k % NRING]
                sem = sems[k % NRING]

                if k >= NRING:
                    pltpu.make_async_copy(
                        ebuf, ebd_hbm.at[pl.ds(0, DH), :], sem).wait()
                else:
                    @pl.when(i > 0)
                    def _():
                        pltpu.make_async_copy(
                            ebuf, ebd_hbm.at[pl.ds(0, DH), :], sem).wait()

                @plsc.parallel_loop(0, N // LANES, step=1)
                def j_loop(j):
                    v = xbuf[xrow + r, pl.ds(j * LANES, LANES)]
                    base = (v * 288.0).astype(jnp.int32) + h * (DH * NT)
                    for d in range(DH):
                        g = plsc.load_gather(tabv, [base + d * NT])
                        ebuf[d, pl.ds(j * LANES, LANES)] = g

                out_row = (bt0 + i * GRP + r) * D + h * DH
                pltpu.async_copy(ebuf, ebd_hbm.at[pl.ds(out_row, DH), :], sem)
        return carry

    lax.fori_loop(0, n_i, i_body, 0)
    for q in range(NRING):
        pltpu.make_async_copy(ebufs[q], ebd_hbm.at[pl.ds(0, DH), :], sems[q]).wait()


def _tc_slice_body(x_ref, o_ref):
    o_ref[...] = x_ref[...]


@jax.jit
def kernel(x, time_table):
    b, n, t, ch = x.shape
    assert (b, n, t, ch) == (B, N, NT, NCH) and time_table.shape == (NT, D)

    xt = jnp.transpose(x, (0, 3, 2, 1))
    x_rows = xt.reshape(b * ch * t, n)
    tab_flat = time_table.T.reshape(NT * D)

    mesh = plsc.VectorSubcoreMesh(core_axis_name="c", subcore_axis_name="s")
    ebd_rows = pl.kernel(
        _sc_body,
        out_type=jax.ShapeDtypeStruct((b * t * D, n), jnp.float32),
        mesh=mesh,
        compiler_params=pltpu.CompilerParams(
            needs_layout_passes=False, use_tc_tiling_on_sc=True),
        scratch_types=[
            pltpu.VMEM((NT * D,), jnp.float32),
            pltpu.VMEM((2 * GRP, N), jnp.float32),
            pltpu.VMEM((DH, N), jnp.float32),
            pltpu.VMEM((DH, N), jnp.float32),
            pltpu.VMEM((DH, N), jnp.float32),
            pltpu.VMEM((DH, N), jnp.float32),
            pltpu.SemaphoreType.DMA,
            pltpu.SemaphoreType.DMA,
            pltpu.SemaphoreType.DMA,
            pltpu.SemaphoreType.DMA,
            pltpu.SemaphoreType.DMA,
        ],
    )(x_rows, tab_flat)

    xo_t = pl.pallas_call(
        _tc_slice_body,
        grid=(b, NKEEP),
        in_specs=[pl.BlockSpec((1, 1, t, n), lambda i, c: (i, c, 0, 0))],
        out_specs=pl.BlockSpec((1, 1, t, n), lambda i, c: (i, c, 0, 0)),
        out_shape=jax.ShapeDtypeStruct((b, NKEEP, t, n), jnp.float32),
    )(xt)

    xo = jnp.transpose(xo_t, (0, 3, 2, 1))
    ebd = jnp.transpose(ebd_rows.reshape(b, t, D, n), (0, 3, 1, 2))
    return xo, ebd
